# baseline (speedup 1.0000x reference)
import functools

import jax
import jax.numpy as jnp
from jax import lax
from jax.experimental import pallas as pl
from jax.experimental.pallas import tpu as pltpu
from jax.experimental.pallas import tpu_sc as plsc

N = 10000
D = 128
E = 320000
UNITS = 128
MAX_NORM = 0.9

NUM_CORES = 2
NUM_SUBCORES = 16
NW = NUM_CORES * NUM_SUBCORES
CHUNK = 128
NUM_CHUNKS = 80
IDX_SEG = 40
EDGES_PER_TILE = NUM_CHUNKS * CHUNK
E_PAD = EDGES_PER_TILE * NW
ROWS_PER_SC = 10112
ZROWS = ROWS_PER_SC // NUM_SUBCORES

ROW_BLOCK = 1000


def _premap_body(x_ref, o_ref):
    v = x_ref[...]
    n0 = jnp.sqrt(jnp.sum(v * v, axis=1, keepdims=True))
    s0 = jnp.where(n0 > 0, jnp.minimum(1.0, MAX_NORM / (n0 + 1e-08)), 1.0)
    xc = v * s0
    nv = jnp.sqrt(jnp.sum(xc * xc, axis=1, keepdims=True))
    nv = jnp.clip(nv, 0.0, 10.0)
    y = jnp.tanh(nv) * xc / (nv + 1e-08)
    ny = jnp.sqrt(jnp.sum(y * y, axis=1, keepdims=True))
    s1 = jnp.where(ny > 0, jnp.minimum(1.0, MAX_NORM / (ny + 1e-08)), 1.0)
    o_ref[...] = y * s1


def _premap(x):
    return pl.pallas_call(
        _premap_body,
        out_shape=jax.ShapeDtypeStruct((N, D), jnp.float32),
        grid=(N // ROW_BLOCK,),
        in_specs=[pl.BlockSpec((ROW_BLOCK, D), lambda i: (i, 0))],
        out_specs=pl.BlockSpec((ROW_BLOCK, D), lambda i: (i, 0)),
    )(x)


def _postmap_body(p0_ref, p1_ref, w_ref, b_ref, o_ref):
    sup = p0_ref[...] + p1_ref[...]
    n0 = jnp.sqrt(jnp.sum(sup * sup, axis=1, keepdims=True))
    s0 = jnp.where(n0 > 0, jnp.minimum(1.0, MAX_NORM / (n0 + 1e-08)), 1.0)
    sc = sup * s0
    ny = jnp.sqrt(jnp.sum(sc * sc, axis=1, keepdims=True))
    ny = jnp.clip(ny, 0.0, 0.999)
    at = 0.5 * jnp.log((1.0 + ny) / (1.0 - ny))
    mapped = at * sc / (ny + 1e-08)
    o_ref[...] = (
        jnp.dot(mapped, w_ref[...], preferred_element_type=jnp.float32,
                precision=jax.lax.Precision.HIGHEST)
        + b_ref[...]
    )


def _postmap(p0, p1, w, b):
    return pl.pallas_call(
        _postmap_body,
        out_shape=jax.ShapeDtypeStruct((N, UNITS), jnp.float32),
        grid=(N // ROW_BLOCK,),
        in_specs=[
            pl.BlockSpec((ROW_BLOCK, D), lambda i: (i, 0)),
            pl.BlockSpec((ROW_BLOCK, D), lambda i: (i, 0)),
            pl.BlockSpec((D, UNITS), lambda i: (0, 0)),
            pl.BlockSpec((1, UNITS), lambda i: (0, 0)),
        ],
        out_specs=pl.BlockSpec((ROW_BLOCK, UNITS), lambda i: (i, 0)),
    )(p0, p1, w, b.reshape(1, UNITS))


def _sc_body(src_hbm, dst_hbm, xproj_hbm, zeros_hbm, out_hbm,
             src_v, dst_v, rows_a, rows_b, sup_sh, sem_a, sem_b):
    c = lax.axis_index("c")
    s = lax.axis_index("s")
    wid = c * NUM_SUBCORES + s

    pltpu.sync_copy(zeros_hbm, sup_sh.at[pl.ds(s * ZROWS, ZROWS)])
    def gather(l, buf, sem):
        return pltpu.async_copy(xproj_hbm.at[src_v.at[l]], buf, sem)

    def drain_and_add(l, buf, sem):
        pltpu.make_async_copy(xproj_hbm.at[src_v.at[l]], buf, sem).wait()
        pltpu.sync_copy(buf, sup_sh.at[dst_v.at[l]], add=True)

    for seg in range(NUM_CHUNKS // IDX_SEG):
        pltpu.sync_copy(src_hbm.at[wid].at[pl.ds(seg * IDX_SEG, IDX_SEG)], src_v)
        pltpu.sync_copy(dst_hbm.at[wid].at[pl.ds(seg * IDX_SEG, IDX_SEG)], dst_v)
        gather(0, rows_a, sem_a)
        if seg == 0:
            plsc.subcore_barrier()

        def pair(g, carry):
            l = 2 * g
            gather(l + 1, rows_b, sem_b)
            drain_and_add(l, rows_a, sem_a)
            gather(l + 2, rows_a, sem_a)
            drain_and_add(l + 1, rows_b, sem_b)
            return carry

        lax.fori_loop(0, IDX_SEG // 2 - 1, pair, 0)
        l = IDX_SEG - 2
        gather(l + 1, rows_b, sem_b)
        drain_and_add(l, rows_a, sem_a)
        drain_and_add(l + 1, rows_b, sem_b)
    plsc.subcore_barrier()

    pltpu.sync_copy(
        sup_sh.at[pl.ds(s * ZROWS, ZROWS)],
        out_hbm.at[c].at[pl.ds(s * ZROWS, ZROWS)],
    )


def _sc_aggregate(src_idx, dst_idx, xproj, zeros):
    mesh = plsc.VectorSubcoreMesh(
        core_axis_name="c", subcore_axis_name="s",
        num_cores=NUM_CORES, num_subcores=NUM_SUBCORES,
    )
    return pl.kernel(
        _sc_body,
        out_type=jax.ShapeDtypeStruct((NUM_CORES, ROWS_PER_SC, D), jnp.float32),
        mesh=mesh,
        scratch_types=[
            pltpu.VMEM((IDX_SEG, CHUNK), jnp.int32),
            pltpu.VMEM((IDX_SEG, CHUNK), jnp.int32),
            pltpu.VMEM((CHUNK, D), jnp.float32),
            pltpu.VMEM((CHUNK, D), jnp.float32),
            pltpu.VMEM_SHARED((ROWS_PER_SC, D), jnp.float32),
            pltpu.SemaphoreType.DMA,
            pltpu.SemaphoreType.DMA,
        ],
    )(src_idx, dst_idx, xproj, zeros)


def kernel(x, edge_index, kernel, bias):
    w = kernel
    xproj = _premap(x)

    dst = edge_index[0]
    src = edge_index[1]
    pad = E_PAD - E
    src_p = jnp.concatenate([src, jnp.zeros((pad,), jnp.int32)])
    dst_p = jnp.concatenate([dst, jnp.full((pad,), N, jnp.int32)])
    src_idx = src_p.reshape(NW, NUM_CHUNKS, CHUNK)
    dst_idx = dst_p.reshape(NW, NUM_CHUNKS, CHUNK)
    zeros = jnp.zeros((ZROWS, D), jnp.float32)

    partials = _sc_aggregate(src_idx, dst_idx, xproj, zeros)
    return _postmap(partials[0, :N], partials[1, :N], w, bias)


# speedup vs baseline: 1.3757x; 1.3757x over previous
import functools

import jax
import jax.numpy as jnp
from jax import lax
from jax.experimental import pallas as pl
from jax.experimental.pallas import tpu as pltpu
from jax.experimental.pallas import tpu_sc as plsc

N = 10000
D = 128
E = 320000
UNITS = 128
MAX_NORM = 0.9

NUM_CORES = 2
NUM_SUBCORES = 16
NW = NUM_CORES * NUM_SUBCORES
CHUNK = 128
NUM_CHUNKS = 79
EDGES_PER_TILE = NUM_CHUNKS * CHUNK
E_PAD = EDGES_PER_TILE * NW
ROWS_PER_SC = 10112
ZROWS = ROWS_PER_SC // NUM_SUBCORES

ROW_BLOCK = 1000


def _premap_body(x_ref, o_ref):
    v = x_ref[...]
    n0 = jnp.sqrt(jnp.sum(v * v, axis=1, keepdims=True))
    s0 = jnp.where(n0 > 0, jnp.minimum(1.0, MAX_NORM / (n0 + 1e-08)), 1.0)
    xc = v * s0
    nv = jnp.sqrt(jnp.sum(xc * xc, axis=1, keepdims=True))
    nv = jnp.clip(nv, 0.0, 10.0)
    y = jnp.tanh(nv) * xc / (nv + 1e-08)
    ny = jnp.sqrt(jnp.sum(y * y, axis=1, keepdims=True))
    s1 = jnp.where(ny > 0, jnp.minimum(1.0, MAX_NORM / (ny + 1e-08)), 1.0)
    o_ref[...] = y * s1


def _premap(x):
    return pl.pallas_call(
        _premap_body,
        out_shape=jax.ShapeDtypeStruct((N, D), jnp.float32),
        grid=(N // ROW_BLOCK,),
        in_specs=[pl.BlockSpec((ROW_BLOCK, D), lambda i: (i, 0))],
        out_specs=pl.BlockSpec((ROW_BLOCK, D), lambda i: (i, 0)),
    )(x)


def _postmap_body(p0_ref, p1_ref, w_ref, b_ref, o_ref):
    sup = p0_ref[...] + p1_ref[...]
    n0 = jnp.sqrt(jnp.sum(sup * sup, axis=1, keepdims=True))
    s0 = jnp.where(n0 > 0, jnp.minimum(1.0, MAX_NORM / (n0 + 1e-08)), 1.0)
    sc = sup * s0
    ny = jnp.sqrt(jnp.sum(sc * sc, axis=1, keepdims=True))
    ny = jnp.clip(ny, 0.0, 0.999)
    at = 0.5 * jnp.log((1.0 + ny) / (1.0 - ny))
    mapped = at * sc / (ny + 1e-08)
    o_ref[...] = (
        jnp.dot(mapped, w_ref[...], preferred_element_type=jnp.float32,
                precision=jax.lax.Precision.HIGHEST)
        + b_ref[...]
    )


def _postmap(p0, p1, w, b):
    return pl.pallas_call(
        _postmap_body,
        out_shape=jax.ShapeDtypeStruct((N, UNITS), jnp.float32),
        grid=(N // ROW_BLOCK,),
        in_specs=[
            pl.BlockSpec((ROW_BLOCK, D), lambda i: (i, 0)),
            pl.BlockSpec((ROW_BLOCK, D), lambda i: (i, 0)),
            pl.BlockSpec((D, UNITS), lambda i: (0, 0)),
            pl.BlockSpec((1, UNITS), lambda i: (0, 0)),
        ],
        out_specs=pl.BlockSpec((ROW_BLOCK, UNITS), lambda i: (i, 0)),
    )(p0, p1, w, b.reshape(1, UNITS))


def _sc_body(src_hbm, dst_hbm, xproj_hbm, zeros_hbm, out_hbm,
             src_v, dst_v, rows_a, sup_sh, sem_a):
    c = lax.axis_index("c")
    s = lax.axis_index("s")
    wid = c * NUM_SUBCORES + s

    pltpu.sync_copy(zeros_hbm, sup_sh.at[pl.ds(s * ZROWS, ZROWS)])
    pltpu.sync_copy(src_hbm.at[wid], src_v)
    pltpu.sync_copy(dst_hbm.at[wid], dst_v)
    plsc.subcore_barrier()

    def step(j, carry):
        pltpu.async_copy(xproj_hbm.at[src_v.at[j]], rows_a, sem_a).wait()
        pltpu.sync_copy(rows_a, sup_sh.at[dst_v.at[j]], add=True)
        return carry

    lax.fori_loop(0, NUM_CHUNKS, step, 0)
    plsc.subcore_barrier()

    pltpu.sync_copy(
        sup_sh.at[pl.ds(s * ZROWS, ZROWS)],
        out_hbm.at[c].at[pl.ds(s * ZROWS, ZROWS)],
    )


def _sc_aggregate(src_idx, dst_idx, xproj, zeros):
    mesh = plsc.VectorSubcoreMesh(
        core_axis_name="c", subcore_axis_name="s",
        num_cores=NUM_CORES, num_subcores=NUM_SUBCORES,
    )
    return pl.kernel(
        _sc_body,
        out_type=jax.ShapeDtypeStruct((NUM_CORES, ROWS_PER_SC, D), jnp.float32),
        mesh=mesh,
        scratch_types=[
            pltpu.VMEM((NUM_CHUNKS, CHUNK), jnp.int32),
            pltpu.VMEM((NUM_CHUNKS, CHUNK), jnp.int32),
            pltpu.VMEM((CHUNK, D), jnp.float32),
            pltpu.VMEM_SHARED((ROWS_PER_SC, D), jnp.float32),
            pltpu.SemaphoreType.DMA,
        ],
    )(src_idx, dst_idx, xproj, zeros)


def kernel(x, edge_index, kernel, bias):
    w = kernel
    xproj = _premap(x)

    dst = edge_index[0]
    src = edge_index[1]
    pad = E_PAD - E
    src_p = jnp.concatenate([src, jnp.zeros((pad,), jnp.int32)])
    dst_p = jnp.concatenate([dst, jnp.full((pad,), N, jnp.int32)])
    src_idx = src_p.reshape(NW, NUM_CHUNKS, CHUNK)
    dst_idx = dst_p.reshape(NW, NUM_CHUNKS, CHUNK)
    zeros = jnp.zeros((ZROWS, D), jnp.float32)

    partials = _sc_aggregate(src_idx, dst_idx, xproj, zeros)
    return _postmap(partials[0, :N], partials[1, :N], w, bias)
